# baseline (speedup 1.0000x reference)
import functools

import jax
import jax.numpy as jnp
from jax.experimental import pallas as pl

TOP_K = 8
N_EXPERTS = 64
N_GROUP = 8
GROUP_SIZE = N_EXPERTS // N_GROUP
TOPK_GROUP = 4
SCALE = 2.5

_NEG_INF = float("-inf")


def _gate_body(w_ref, x_ref, bias_ref, idx_ref, w_out_ref):
    w = w_ref[...]
    x = x_ref[...]
    logits_t = jax.lax.dot_general(
        w, x, (((1,), (1,)), ((), ())),
        preferred_element_type=jnp.float32,
    )
    scores = 1.0 / (1.0 + jnp.exp(-logits_t))
    biased = scores + bias_ref[...]
    tb = x.shape[0]

    a3 = biased.reshape(N_GROUP, GROUP_SIZE, tb)
    m1 = jnp.max(a3, axis=1, keepdims=True)
    m2 = jnp.max(jnp.where(a3 < m1, a3, _NEG_INF), axis=1,
                 keepdims=True)
    work = m1 + m2

    keep = jnp.zeros_like(work, dtype=jnp.bool_)
    for _ in range(TOPK_GROUP):
        gm = jnp.max(work, axis=0, keepdims=True)
        sel = work == gm
        keep = keep | sel
        work = jnp.where(sel, _NEG_INF, work)

    keep64 = jnp.broadcast_to(keep, (N_GROUP, GROUP_SIZE, tb))
    masked = jnp.where(keep64, a3, 0.0).reshape(N_EXPERTS, tb)

    eid = jax.lax.broadcasted_iota(jnp.int32, (N_EXPERTS, tb), 0)
    idx_rows, w_rows = [], []
    for _ in range(TOP_K):
        m = jnp.max(masked, axis=0, keepdims=True)
        sel = masked == m
        i = jnp.min(jnp.where(sel, eid, N_EXPERTS), axis=0,
                    keepdims=True)
        picked = eid == i
        w_rows.append(jnp.max(jnp.where(picked, scores, _NEG_INF),
                              axis=0, keepdims=True))
        masked = jnp.where(picked, _NEG_INF, masked)
        idx_rows.append(i)

    idx_t = jnp.concatenate(idx_rows, axis=0)
    w_t = jnp.concatenate(w_rows, axis=0)
    denom = jnp.sum(w_t, axis=0, keepdims=True) + 1e-20
    idx_ref[...] = idx_t.T
    w_out_ref[...] = ((w_t / denom) * SCALE).T


@functools.partial(jax.jit, static_argnames=("block_tokens", "interpret"))
def _gate(x, w, bias, block_tokens=2048, interpret=False):
    t, h = x.shape
    grid = (t // block_tokens,)
    return pl.pallas_call(
        _gate_body,
        grid=grid,
        in_specs=[
            pl.BlockSpec((N_EXPERTS, h), lambda i: (0, 0)),
            pl.BlockSpec((block_tokens, h), lambda i: (i, 0)),
            pl.BlockSpec((N_EXPERTS, 1), lambda i: (0, 0)),
        ],
        out_specs=[
            pl.BlockSpec((block_tokens, TOP_K), lambda i: (i, 0)),
            pl.BlockSpec((block_tokens, TOP_K), lambda i: (i, 0)),
        ],
        out_shape=[
            jax.ShapeDtypeStruct((t, TOP_K), jnp.int32),
            jax.ShapeDtypeStruct((t, TOP_K), jnp.float32),
        ],
        interpret=interpret,
    )(w, x, bias)


def kernel(hidden_states, weight, e_score_correction_bias):
    bsz, seq, hidden = hidden_states.shape
    x = hidden_states.reshape(bsz * seq, hidden).astype(jnp.float32)
    w = weight.astype(jnp.float32)
    bias = e_score_correction_bias.astype(jnp.float32).reshape(N_EXPERTS, 1)
    return _gate(x, w, bias)


# speedup vs baseline: 1.3158x; 1.3158x over previous
import functools

import jax
import jax.numpy as jnp
from jax.experimental import pallas as pl

TOP_K = 8
N_EXPERTS = 64
N_GROUP = 8
GROUP_SIZE = N_EXPERTS // N_GROUP
TOPK_GROUP = 4
SCALE = 2.5

_NEG_INF = float("-inf")


def _gate_body(w_ref, x_ref, bias_ref, idx_ref, w_out_ref):
    w = w_ref[...]
    x = x_ref[...]
    logits_t = jax.lax.dot_general(
        w, x, (((1,), (1,)), ((), ())),
        preferred_element_type=jnp.float32,
    )
    scores = 1.0 / (1.0 + jnp.exp(-logits_t))
    biased = scores + bias_ref[...]
    tb = x.shape[0]

    a3 = biased.reshape(N_GROUP, GROUP_SIZE, tb)
    sub_iota = jax.lax.broadcasted_iota(jnp.int32,
                                        (N_GROUP, GROUP_SIZE, tb), 1)
    m1 = jnp.max(a3, axis=1, keepdims=True)
    i1 = jnp.min(jnp.where(a3 == m1, sub_iota, GROUP_SIZE),
                 axis=1, keepdims=True)
    m2 = jnp.max(jnp.where(sub_iota == i1, _NEG_INF, a3), axis=1,
                 keepdims=True)
    work = m1 + m2

    g_iota = jax.lax.broadcasted_iota(jnp.int32, (N_GROUP, 1, tb), 0)
    keep = jnp.zeros_like(work, dtype=jnp.bool_)
    for _ in range(TOPK_GROUP):
        gm = jnp.max(work, axis=0, keepdims=True)
        jg = jnp.min(jnp.where(work == gm, g_iota, N_GROUP),
                     axis=0, keepdims=True)
        sel = g_iota == jg
        keep = keep | sel
        work = jnp.where(sel, _NEG_INF, work)

    keep64 = jnp.broadcast_to(keep, (N_GROUP, GROUP_SIZE, tb))
    masked = jnp.where(keep64, a3, 0.0).reshape(N_EXPERTS, tb)

    eid = jax.lax.broadcasted_iota(jnp.int32, (N_EXPERTS, tb), 0)
    idx_rows, w_rows = [], []
    for _ in range(TOP_K):
        m = jnp.max(masked, axis=0, keepdims=True)
        sel = masked == m
        i = jnp.min(jnp.where(sel, eid, N_EXPERTS), axis=0,
                    keepdims=True)
        picked = eid == i
        w_rows.append(jnp.max(jnp.where(picked, scores, _NEG_INF),
                              axis=0, keepdims=True))
        masked = jnp.where(picked, _NEG_INF, masked)
        idx_rows.append(i)

    idx_t = jnp.concatenate(idx_rows, axis=0)
    w_t = jnp.concatenate(w_rows, axis=0)
    denom = jnp.sum(w_t, axis=0, keepdims=True) + 1e-20
    idx_ref[...] = idx_t
    w_out_ref[...] = (w_t / denom) * SCALE


@functools.partial(jax.jit, static_argnames=("block_tokens", "interpret"))
def _gate(x, w, bias, block_tokens=2048, interpret=False):
    t, h = x.shape
    grid = (t // block_tokens,)
    return pl.pallas_call(
        _gate_body,
        grid=grid,
        in_specs=[
            pl.BlockSpec((N_EXPERTS, h), lambda i: (0, 0)),
            pl.BlockSpec((block_tokens, h), lambda i: (i, 0)),
            pl.BlockSpec((N_EXPERTS, 1), lambda i: (0, 0)),
        ],
        out_specs=[
            pl.BlockSpec((TOP_K, block_tokens), lambda i: (0, i)),
            pl.BlockSpec((TOP_K, block_tokens), lambda i: (0, i)),
        ],
        out_shape=[
            jax.ShapeDtypeStruct((TOP_K, t), jnp.int32),
            jax.ShapeDtypeStruct((TOP_K, t), jnp.float32),
        ],
        interpret=interpret,
    )(w, x, bias)


def kernel(hidden_states, weight, e_score_correction_bias):
    bsz, seq, hidden = hidden_states.shape
    x = hidden_states.reshape(bsz * seq, hidden).astype(jnp.float32)
    w = weight.astype(jnp.float32)
    bias = e_score_correction_bias.astype(jnp.float32).reshape(N_EXPERTS, 1)
    idx_t, w_t = _gate(x, w, bias)
    return idx_t.T, w_t.T
